# Initial kernel scaffold; baseline (speedup 1.0000x reference)
import jax
import jax.numpy as jnp
from jax.experimental import pallas as pl


def kernel(h, edges, coords, edge_attr, params):
    raise NotImplementedError("write your pallas kernel here")


# speedup vs baseline: 2.4342x; 2.4342x over previous
import functools

import jax
import jax.numpy as jnp
from jax import lax
from jax.experimental import pallas as pl
from jax.experimental.pallas import tpu as pltpu
from jax.experimental.pallas import tpu_sc as plsc

N = 10000
E = 320000
D = 128
CD = 16
ED = 16

NC = 2
NS = 16
NW = NC * NS
EPW = E // NW
ECHUNK = 80
NCHUNKS = EPW // ECHUNK
NP = 10240
NPT = NP // NS
ZROWS = 128

BN = 400
BE = 512


def _silu(x):
    return x * jax.nn.sigmoid(x)


def _tables_body(h_ref, c_ref, w1s_ref, w1d_ref, wr_ref, b1_ref,
                 ts_ref, td_ref):
    h = h_ref[...]
    c = c_ref[...]
    rw = jnp.sum(c * c, axis=1, keepdims=True) * wr_ref[...]
    ts_ref[...] = jnp.dot(h, w1s_ref[...], preferred_element_type=jnp.float32) + rw
    td_ref[...] = (jnp.dot(h, w1d_ref[...], preferred_element_type=jnp.float32)
                   + rw + b1_ref[...])


def _tables(h, cpad, w1s, w1d, wr, b1):
    return pl.pallas_call(
        _tables_body,
        grid=(N // BN,),
        in_specs=[
            pl.BlockSpec((BN, D), lambda i: (i, 0)),
            pl.BlockSpec((BN, CD), lambda i: (i, 0)),
            pl.BlockSpec((D, D), lambda i: (0, 0)),
            pl.BlockSpec((D, D), lambda i: (0, 0)),
            pl.BlockSpec((1, D), lambda i: (0, 0)),
            pl.BlockSpec((1, D), lambda i: (0, 0)),
        ],
        out_specs=[pl.BlockSpec((BN, D), lambda i: (i, 0))] * 2,
        out_shape=[jax.ShapeDtypeStruct((N, D), jnp.float32)] * 2,
    )(h, cpad, w1s, w1d, wr, b1)


def _edge_body(g1_ref, g2_ref, cr_ref, ea_ref, w1e_ref, wr_ref,
               w2_ref, b2_ref, m_ref):
    pre = (g1_ref[...] + g2_ref[...]
           + jnp.dot(ea_ref[...], w1e_ref[...], preferred_element_type=jnp.float32)
           + cr_ref[...] * wr_ref[...])
    m_ref[...] = _silu(
        jnp.dot(_silu(pre), w2_ref[...], preferred_element_type=jnp.float32)
        + b2_ref[...])


def _edge(g1, g2, cross, ea, w1e, wr, w2, b2):
    return pl.pallas_call(
        _edge_body,
        grid=(E // BE,),
        in_specs=[
            pl.BlockSpec((BE, D), lambda i: (i, 0)),
            pl.BlockSpec((BE, D), lambda i: (i, 0)),
            pl.BlockSpec((BE, 1), lambda i: (i, 0)),
            pl.BlockSpec((BE, ED), lambda i: (i, 0)),
            pl.BlockSpec((ED, D), lambda i: (0, 0)),
            pl.BlockSpec((1, D), lambda i: (0, 0)),
            pl.BlockSpec((D, D), lambda i: (0, 0)),
            pl.BlockSpec((1, D), lambda i: (0, 0)),
        ],
        out_specs=pl.BlockSpec((BE, D), lambda i: (i, 0)),
        out_shape=jax.ShapeDtypeStruct((E, D), jnp.float32),
    )(g1, g2, cross, ea, w1e, wr, w2, b2)


def _node_body(h_ref, agg_ref, w1_ref, b1_ref, w2_ref, b2_ref, o_ref,
               *, residual):
    h = h_ref[...]
    a = agg_ref[0] + agg_ref[1]
    z = (jnp.dot(h, w1_ref[:D], preferred_element_type=jnp.float32)
         + jnp.dot(a, w1_ref[D:], preferred_element_type=jnp.float32)
         + b1_ref[...])
    o = (jnp.dot(_silu(z), w2_ref[...], preferred_element_type=jnp.float32)
         + b2_ref[...])
    o_ref[...] = o + h if residual else o


def _node(h, agg2, w1, b1, w2, b2, residual):
    return pl.pallas_call(
        functools.partial(_node_body, residual=residual),
        grid=(N // BN,),
        in_specs=[
            pl.BlockSpec((BN, D), lambda i: (i, 0)),
            pl.BlockSpec((2, BN, D), lambda i: (0, i, 0)),
            pl.BlockSpec((2 * D, D), lambda i: (0, 0)),
            pl.BlockSpec((1, D), lambda i: (0, 0)),
            pl.BlockSpec((D, D), lambda i: (0, 0)),
            pl.BlockSpec((1, D), lambda i: (0, 0)),
        ],
        out_specs=pl.BlockSpec((BN, D), lambda i: (i, 0)),
        out_shape=jax.ShapeDtypeStruct((N, D), jnp.float32),
    )(h, agg2, w1, b1, w2, b2)


_MESH = plsc.VectorSubcoreMesh(core_axis_name="c", subcore_axis_name="s")


@functools.partial(
    pl.kernel,
    mesh=_MESH,
    out_type=(
        jax.ShapeDtypeStruct((E, D), jnp.float32),
        jax.ShapeDtypeStruct((E, D), jnp.float32),
        jax.ShapeDtypeStruct((E,), jnp.float32),
    ),
    scratch_types=[
        pltpu.VMEM((ECHUNK,), jnp.int32),
        pltpu.VMEM((ECHUNK,), jnp.int32),
        pltpu.VMEM((ECHUNK, D), jnp.float32),
        pltpu.VMEM((ECHUNK, D), jnp.float32),
        pltpu.VMEM((ECHUNK,), jnp.float32),
        pltpu.VMEM((N,), jnp.float32),
        pltpu.VMEM((N,), jnp.float32),
        pltpu.VMEM((N,), jnp.float32),
        pltpu.SemaphoreType.DMA,
    ],
    compiler_params=pltpu.CompilerParams(needs_layout_passes=False),
)
def _gather(ts_hbm, td_hbm, xs_hbm, ys_hbm, zs_hbm, row_hbm, col_hbm,
            g1_hbm, g2_hbm, cr_hbm,
            ir_v, ic_v, b1_v, b2_v, cr_v, xs_v, ys_v, zs_v, sem):
    wid = lax.axis_index("s") * NC + lax.axis_index("c")
    base = wid * EPW

    pltpu.sync_copy(xs_hbm, xs_v)
    pltpu.sync_copy(ys_hbm, ys_v)
    pltpu.sync_copy(zs_hbm, zs_v)

    def chunk(i, carry):
        off = base + i * ECHUNK
        pltpu.sync_copy(row_hbm.at[pl.ds(off, ECHUNK)], ir_v)
        pltpu.sync_copy(col_hbm.at[pl.ds(off, ECHUNK)], ic_v)
        d1 = pltpu.async_copy(ts_hbm.at[ir_v], b1_v, sem)
        d2 = pltpu.async_copy(td_hbm.at[ic_v], b2_v, sem)
        for j in range(ECHUNK // 16):
            ii = ir_v[pl.ds(j * 16, 16)]
            jj = ic_v[pl.ds(j * 16, 16)]
            dot = (plsc.load_gather(xs_v, [ii]) * plsc.load_gather(xs_v, [jj])
                   + plsc.load_gather(ys_v, [ii]) * plsc.load_gather(ys_v, [jj])
                   + plsc.load_gather(zs_v, [ii]) * plsc.load_gather(zs_v, [jj]))
            cr_v[pl.ds(j * 16, 16)] = -2.0 * dot
        d1.wait()
        d2.wait()
        pltpu.sync_copy(b1_v, g1_hbm.at[pl.ds(off, ECHUNK)])
        pltpu.sync_copy(b2_v, g2_hbm.at[pl.ds(off, ECHUNK)])
        pltpu.sync_copy(cr_v, cr_hbm.at[pl.ds(off, ECHUNK)])
        return carry

    lax.fori_loop(0, NCHUNKS, chunk, 0)


@functools.partial(
    pl.kernel,
    mesh=_MESH,
    out_type=jax.ShapeDtypeStruct((NC, NP, D), jnp.float32),
    scratch_types=[
        pltpu.VMEM((ECHUNK,), jnp.int32),
        pltpu.VMEM((ECHUNK, D), jnp.float32),
        pltpu.VMEM((ZROWS, D), jnp.float32),
        pltpu.VMEM_SHARED((NP, D), jnp.float32),
        pltpu.SemaphoreType.DMA,
    ],
)
def _scatter(m_hbm, row_hbm, out_hbm, idx_v, mb_v, zb_v, acc_sh, sem):
    cid = lax.axis_index("c")
    sid = lax.axis_index("s")
    wid = sid * NC + cid
    tbase = sid * NPT

    zero = jnp.zeros((16,), jnp.float32)

    def zrow(r, carry):
        def zcol(c, carry2):
            zb_v[r, pl.ds(c * 16, 16)] = zero
            return carry2
        return lax.fori_loop(0, D // 16, zcol, carry)

    lax.fori_loop(0, ZROWS, zrow, 0)

    def zcopy(j, carry):
        pltpu.sync_copy(zb_v, acc_sh.at[pl.ds(tbase + j * ZROWS, ZROWS)])
        return carry

    lax.fori_loop(0, NPT // ZROWS, zcopy, 0)
    plsc.subcore_barrier()

    base = wid * EPW

    def chunk(i, carry):
        off = base + i * ECHUNK
        pltpu.sync_copy(row_hbm.at[pl.ds(off, ECHUNK)], idx_v)
        pltpu.sync_copy(m_hbm.at[pl.ds(off, ECHUNK)], mb_v)
        pltpu.sync_copy(mb_v, acc_sh.at[idx_v], add=True)
        return carry

    lax.fori_loop(0, NCHUNKS, chunk, 0)
    plsc.subcore_barrier()

    pltpu.sync_copy(acc_sh.at[pl.ds(tbase, NPT)],
                    out_hbm.at[cid, pl.ds(tbase, NPT)])


def kernel(h, edges, coords, edge_attr, params):
    row, col = edges[0], edges[1]
    cpad = jnp.pad(coords, ((0, 0), (0, CD - 3)))
    xs, ys, zs = coords[:, 0], coords[:, 1], coords[:, 2]
    out = h
    for i, p in enumerate(params):
        w1 = p["edge_W1"]
        w1s, w1d = w1[:D], w1[D:2 * D]
        wr = w1[2 * D:2 * D + 1]
        w1e = w1[2 * D + 1:]
        b1 = p["edge_b1"][None, :]
        b2 = p["edge_b2"][None, :]
        ts, td = _tables(out, cpad, w1s, w1d, wr, b1)
        g1, g2, cross = _gather(ts, td, xs, ys, zs, row, col)
        m = _edge(g1, g2, cross[:, None], edge_attr, w1e, wr,
                  p["edge_W2"], b2)
        agg2 = _scatter(m, row)
        out = _node(out, agg2, p["node_W1"], p["node_b1"][None, :],
                    p["node_W2"], p["node_b2"][None, :], residual=(i > 0))
    return out


# speedup vs baseline: 3.1666x; 1.3009x over previous
import functools

import jax
import jax.numpy as jnp
from jax import lax
from jax.experimental import pallas as pl
from jax.experimental.pallas import tpu as pltpu
from jax.experimental.pallas import tpu_sc as plsc

N = 10000
E = 320000
D = 128
CD = 16
ED = 16

NC = 2
NS = 16
NW = NC * NS
EPW = E // NW
ECHUNK = 80
NCHUNKS = EPW // ECHUNK
NP = 10240
NPT = NP // NS
ZROWS = 128

BN = 400
BE = 512


def _silu(x):
    return x * jax.nn.sigmoid(x)


def _tables_body(h_ref, c_ref, w1s_ref, w1d_ref, wr_ref, b1_ref,
                 ts_ref, td_ref):
    h = h_ref[...]
    c = c_ref[...]
    rw = jnp.sum(c * c, axis=1, keepdims=True) * wr_ref[...]
    ts_ref[...] = jnp.dot(h, w1s_ref[...], preferred_element_type=jnp.float32) + rw
    td_ref[...] = (jnp.dot(h, w1d_ref[...], preferred_element_type=jnp.float32)
                   + rw + b1_ref[...])


def _tables(h, cpad, w1s, w1d, wr, b1):
    return pl.pallas_call(
        _tables_body,
        grid=(N // BN,),
        in_specs=[
            pl.BlockSpec((BN, D), lambda i: (i, 0)),
            pl.BlockSpec((BN, CD), lambda i: (i, 0)),
            pl.BlockSpec((D, D), lambda i: (0, 0)),
            pl.BlockSpec((D, D), lambda i: (0, 0)),
            pl.BlockSpec((1, D), lambda i: (0, 0)),
            pl.BlockSpec((1, D), lambda i: (0, 0)),
        ],
        out_specs=[pl.BlockSpec((BN, D), lambda i: (i, 0))] * 2,
        out_shape=[jax.ShapeDtypeStruct((N, D), jnp.float32)] * 2,
    )(h, cpad, w1s, w1d, wr, b1)


def _edge_body(g1_ref, g2_ref, cr_ref, ea_ref, w1e_ref, wr_ref,
               w2_ref, b2_ref, m_ref):
    pre = (g1_ref[...] + g2_ref[...]
           + jnp.dot(ea_ref[...], w1e_ref[...], preferred_element_type=jnp.float32)
           + cr_ref[...] * wr_ref[...])
    m_ref[...] = _silu(
        jnp.dot(_silu(pre), w2_ref[...], preferred_element_type=jnp.float32)
        + b2_ref[...])


def _edge(g1, g2, cross, ea, w1e, wr, w2, b2):
    return pl.pallas_call(
        _edge_body,
        grid=(E // BE,),
        in_specs=[
            pl.BlockSpec((BE, D), lambda i: (i, 0)),
            pl.BlockSpec((BE, D), lambda i: (i, 0)),
            pl.BlockSpec((BE, 1), lambda i: (i, 0)),
            pl.BlockSpec((BE, ED), lambda i: (i, 0)),
            pl.BlockSpec((ED, D), lambda i: (0, 0)),
            pl.BlockSpec((1, D), lambda i: (0, 0)),
            pl.BlockSpec((D, D), lambda i: (0, 0)),
            pl.BlockSpec((1, D), lambda i: (0, 0)),
        ],
        out_specs=pl.BlockSpec((BE, D), lambda i: (i, 0)),
        out_shape=jax.ShapeDtypeStruct((E, D), jnp.float32),
    )(g1, g2, cross, ea, w1e, wr, w2, b2)


def _node_body(h_ref, agg_ref, w1_ref, b1_ref, w2_ref, b2_ref, o_ref,
               *, residual):
    h = h_ref[...]
    a = agg_ref[0] + agg_ref[1]
    z = (jnp.dot(h, w1_ref[:D], preferred_element_type=jnp.float32)
         + jnp.dot(a, w1_ref[D:], preferred_element_type=jnp.float32)
         + b1_ref[...])
    o = (jnp.dot(_silu(z), w2_ref[...], preferred_element_type=jnp.float32)
         + b2_ref[...])
    o_ref[...] = o + h if residual else o


def _node(h, agg2, w1, b1, w2, b2, residual):
    return pl.pallas_call(
        functools.partial(_node_body, residual=residual),
        grid=(N // BN,),
        in_specs=[
            pl.BlockSpec((BN, D), lambda i: (i, 0)),
            pl.BlockSpec((2, BN, D), lambda i: (0, i, 0)),
            pl.BlockSpec((2 * D, D), lambda i: (0, 0)),
            pl.BlockSpec((1, D), lambda i: (0, 0)),
            pl.BlockSpec((D, D), lambda i: (0, 0)),
            pl.BlockSpec((1, D), lambda i: (0, 0)),
        ],
        out_specs=pl.BlockSpec((BN, D), lambda i: (i, 0)),
        out_shape=jax.ShapeDtypeStruct((N, D), jnp.float32),
    )(h, agg2, w1, b1, w2, b2)


_MESH = plsc.VectorSubcoreMesh(core_axis_name="c", subcore_axis_name="s")


@functools.partial(
    pl.kernel,
    mesh=_MESH,
    out_type=(
        jax.ShapeDtypeStruct((E, D), jnp.float32),
        jax.ShapeDtypeStruct((E, D), jnp.float32),
        jax.ShapeDtypeStruct((E,), jnp.float32),
    ),
    scratch_types=[
        pltpu.VMEM((EPW,), jnp.int32),
        pltpu.VMEM((EPW,), jnp.int32),
        pltpu.VMEM((EPW,), jnp.float32),
        pltpu.VMEM((N,), jnp.float32),
        pltpu.VMEM((N,), jnp.float32),
        pltpu.VMEM((N,), jnp.float32),
        pltpu.VMEM((ECHUNK, D), jnp.float32),
        pltpu.VMEM((ECHUNK, D), jnp.float32),
        pltpu.VMEM((ECHUNK, D), jnp.float32),
        pltpu.VMEM((ECHUNK, D), jnp.float32),
        pltpu.SemaphoreType.DMA,
        pltpu.SemaphoreType.DMA,
        pltpu.SemaphoreType.DMA,
        pltpu.SemaphoreType.DMA,
    ],
    compiler_params=pltpu.CompilerParams(needs_layout_passes=False),
)
def _gather(ts_hbm, td_hbm, xs_hbm, ys_hbm, zs_hbm, row_hbm, col_hbm,
            g1_hbm, g2_hbm, cr_hbm,
            ir_all, ic_all, cr_all, xs_v, ys_v, zs_v,
            b1a, b2a, b1b, b2b, sg0, sg1, sw0, sw1):
    wid = lax.axis_index("s") * NC + lax.axis_index("c")
    base = wid * EPW

    pltpu.sync_copy(row_hbm.at[pl.ds(base, EPW)], ir_all)
    pltpu.sync_copy(col_hbm.at[pl.ds(base, EPW)], ic_all)
    pltpu.sync_copy(xs_hbm, xs_v)
    pltpu.sync_copy(ys_hbm, ys_v)
    pltpu.sync_copy(zs_hbm, zs_v)

    def issue(c, b1, b2, sem):
        off = c * ECHUNK
        pltpu.async_copy(ts_hbm.at[ir_all.at[pl.ds(off, ECHUNK)]], b1, sem)
        pltpu.async_copy(td_hbm.at[ic_all.at[pl.ds(off, ECHUNK)]], b2, sem)

    def wait_g(b1, b2, sem):
        pltpu.make_async_copy(ts_hbm.at[pl.ds(0, ECHUNK)], b1, sem).wait()
        pltpu.make_async_copy(td_hbm.at[pl.ds(0, ECHUNK)], b2, sem).wait()

    def wb(c, b1, b2, sem):
        off = base + c * ECHUNK
        pltpu.async_copy(b1, g1_hbm.at[pl.ds(off, ECHUNK)], sem)
        pltpu.async_copy(b2, g2_hbm.at[pl.ds(off, ECHUNK)], sem)

    def wait_wb(b1, b2, sem):
        pltpu.make_async_copy(b1, g1_hbm.at[pl.ds(0, ECHUNK)], sem).wait()
        pltpu.make_async_copy(b2, g2_hbm.at[pl.ds(0, ECHUNK)], sem).wait()

    def cross(c):
        coff = c * ECHUNK
        for j in range(ECHUNK // 16):
            ii = ir_all[pl.ds(coff + j * 16, 16)]
            jj = ic_all[pl.ds(coff + j * 16, 16)]
            dot = (plsc.load_gather(xs_v, [ii]) * plsc.load_gather(xs_v, [jj])
                   + plsc.load_gather(ys_v, [ii]) * plsc.load_gather(ys_v, [jj])
                   + plsc.load_gather(zs_v, [ii]) * plsc.load_gather(zs_v, [jj]))
            cr_all[pl.ds(coff + j * 16, 16)] = -2.0 * dot

    issue(0, b1a, b2a, sg0)

    def body(j, carry):
        a = 2 * j
        b = a + 1

        @pl.when(j > 0)
        def _():
            wait_wb(b1b, b2b, sw1)
        issue(b, b1b, b2b, sg1)
        wait_g(b1a, b2a, sg0)
        cross(a)
        wb(a, b1a, b2a, sw0)
        wait_wb(b1a, b2a, sw0)
        issue(a + 2, b1a, b2a, sg0)
        wait_g(b1b, b2b, sg1)
        cross(b)
        wb(b, b1b, b2b, sw1)
        return carry

    lax.fori_loop(0, (NCHUNKS - 1) // 2, body, 0)

    last = NCHUNKS - 1
    wait_g(b1a, b2a, sg0)
    cross(last)
    wb(last, b1a, b2a, sw0)
    wait_wb(b1a, b2a, sw0)
    wait_wb(b1b, b2b, sw1)
    pltpu.sync_copy(cr_all, cr_hbm.at[pl.ds(base, EPW)])


@functools.partial(
    pl.kernel,
    mesh=_MESH,
    out_type=jax.ShapeDtypeStruct((NC, NP, D), jnp.float32),
    scratch_types=[
        pltpu.VMEM((ECHUNK,), jnp.int32),
        pltpu.VMEM((ECHUNK,), jnp.int32),
        pltpu.VMEM((ECHUNK, D), jnp.float32),
        pltpu.VMEM((ECHUNK, D), jnp.float32),
        pltpu.VMEM((ZROWS, D), jnp.float32),
        pltpu.VMEM_SHARED((NP, D), jnp.float32),
        pltpu.SemaphoreType.DMA,
        pltpu.SemaphoreType.DMA,
    ],
)
def _scatter(m_hbm, row_hbm, out_hbm, i0, i1, m0, m1, zb_v, acc_sh, s0, s1):
    cid = lax.axis_index("c")
    sid = lax.axis_index("s")
    wid = sid * NC + cid
    tbase = sid * NPT

    zero = jnp.zeros((16,), jnp.float32)

    def zrow(r, carry):
        def zcol(c, carry2):
            zb_v[r, pl.ds(c * 16, 16)] = zero
            return carry2
        return lax.fori_loop(0, D // 16, zcol, carry)

    lax.fori_loop(0, ZROWS, zrow, 0)

    def zcopy(j, carry):
        pltpu.sync_copy(zb_v, acc_sh.at[pl.ds(tbase + j * ZROWS, ZROWS)])
        return carry

    lax.fori_loop(0, NPT // ZROWS, zcopy, 0)
    plsc.subcore_barrier()

    base = wid * EPW

    def issue(c, iv, mv, sem):
        off = base + c * ECHUNK
        pltpu.async_copy(row_hbm.at[pl.ds(off, ECHUNK)], iv, sem)
        pltpu.async_copy(m_hbm.at[pl.ds(off, ECHUNK)], mv, sem)

    def wait_ld(iv, mv, sem):
        pltpu.make_async_copy(row_hbm.at[pl.ds(0, ECHUNK)], iv, sem).wait()
        pltpu.make_async_copy(m_hbm.at[pl.ds(0, ECHUNK)], mv, sem).wait()

    issue(0, i0, m0, s0)

    def body(j, carry):
        issue(2 * j + 1, i1, m1, s1)
        wait_ld(i0, m0, s0)
        pltpu.sync_copy(m0, acc_sh.at[i0], add=True)
        issue(2 * j + 2, i0, m0, s0)
        wait_ld(i1, m1, s1)
        pltpu.sync_copy(m1, acc_sh.at[i1], add=True)
        return carry

    lax.fori_loop(0, (NCHUNKS - 1) // 2, body, 0)
    wait_ld(i0, m0, s0)
    pltpu.sync_copy(m0, acc_sh.at[i0], add=True)
    plsc.subcore_barrier()

    pltpu.sync_copy(acc_sh.at[pl.ds(tbase, NPT)],
                    out_hbm.at[cid, pl.ds(tbase, NPT)])


def kernel(h, edges, coords, edge_attr, params):
    row, col = edges[0], edges[1]
    cpad = jnp.pad(coords, ((0, 0), (0, CD - 3)))
    xs, ys, zs = coords[:, 0], coords[:, 1], coords[:, 2]
    out = h
    for i, p in enumerate(params):
        w1 = p["edge_W1"]
        w1s, w1d = w1[:D], w1[D:2 * D]
        wr = w1[2 * D:2 * D + 1]
        w1e = w1[2 * D + 1:]
        b1 = p["edge_b1"][None, :]
        b2 = p["edge_b2"][None, :]
        ts, td = _tables(out, cpad, w1s, w1d, wr, b1)
        g1, g2, cross = _gather(ts, td, xs, ys, zs, row, col)
        m = _edge(g1, g2, cross[:, None], edge_attr, w1e, wr,
                  p["edge_W2"], b2)
        agg2 = _scatter(m, row)
        out = _node(out, agg2, p["node_W1"], p["node_b1"][None, :],
                    p["node_W2"], p["node_b2"][None, :], residual=(i > 0))
    return out


# speedup vs baseline: 3.4487x; 1.0891x over previous
import functools

import jax
import jax.numpy as jnp
from jax import lax
from jax.experimental import pallas as pl
from jax.experimental.pallas import tpu as pltpu
from jax.experimental.pallas import tpu_sc as plsc

N = 10000
E = 320000
D = 128
CD = 16
ED = 16

NC = 2
NS = 16
NW = NC * NS
NSEG = 5
ESEG = E // NSEG
EPW = ESEG // NW
ECHUNK = 80
NCHUNKS = EPW // ECHUNK
NP = 10240
NPT = NP // NS
ZROWS = 128

BN = 400
BE = 512


def _silu(x):
    return x * jax.nn.sigmoid(x)


def _tables_body(h_ref, c_ref, w1s_ref, w1d_ref, wr_ref, b1_ref,
                 ts_ref, td_ref):
    h = h_ref[...]
    c = c_ref[...]
    rw = jnp.sum(c * c, axis=1, keepdims=True) * wr_ref[...]
    ts_ref[...] = jnp.dot(h, w1s_ref[...], preferred_element_type=jnp.float32) + rw
    td_ref[...] = (jnp.dot(h, w1d_ref[...], preferred_element_type=jnp.float32)
                   + rw + b1_ref[...])


def _tables(h, cpad, w1s, w1d, wr, b1):
    return pl.pallas_call(
        _tables_body,
        grid=(N // BN,),
        in_specs=[
            pl.BlockSpec((BN, D), lambda i: (i, 0)),
            pl.BlockSpec((BN, CD), lambda i: (i, 0)),
            pl.BlockSpec((D, D), lambda i: (0, 0)),
            pl.BlockSpec((D, D), lambda i: (0, 0)),
            pl.BlockSpec((1, D), lambda i: (0, 0)),
            pl.BlockSpec((1, D), lambda i: (0, 0)),
        ],
        out_specs=[pl.BlockSpec((BN, D), lambda i: (i, 0))] * 2,
        out_shape=[jax.ShapeDtypeStruct((N, D), jnp.float32)] * 2,
    )(h, cpad, w1s, w1d, wr, b1)


def _edge_body(g1_ref, g2_ref, cr_ref, ea_ref, w1e_ref, wr_ref,
               w2_ref, b2_ref, m_ref):
    pre = (g1_ref[...] + g2_ref[...]
           + jnp.dot(ea_ref[...], w1e_ref[...], preferred_element_type=jnp.float32)
           + cr_ref[...] * wr_ref[...])
    m_ref[...] = _silu(
        jnp.dot(_silu(pre), w2_ref[...], preferred_element_type=jnp.float32)
        + b2_ref[...])


def _edge(g1, g2, cross, ea, w1e, wr, w2, b2):
    return pl.pallas_call(
        _edge_body,
        grid=(ESEG // BE,),
        in_specs=[
            pl.BlockSpec((BE, D), lambda i: (i, 0)),
            pl.BlockSpec((BE, D), lambda i: (i, 0)),
            pl.BlockSpec((BE, 1), lambda i: (i, 0)),
            pl.BlockSpec((BE, ED), lambda i: (i, 0)),
            pl.BlockSpec((ED, D), lambda i: (0, 0)),
            pl.BlockSpec((1, D), lambda i: (0, 0)),
            pl.BlockSpec((D, D), lambda i: (0, 0)),
            pl.BlockSpec((1, D), lambda i: (0, 0)),
        ],
        out_specs=pl.BlockSpec((BE, D), lambda i: (i, 0)),
        out_shape=jax.ShapeDtypeStruct((ESEG, D), jnp.float32),
    )(g1, g2, cross, ea, w1e, wr, w2, b2)


def _node_body(h_ref, *rest, residual):
    parts = rest[:NSEG]
    w1_ref, b1_ref, w2_ref, b2_ref, o_ref = rest[NSEG:]
    h = h_ref[...]
    a = parts[0][0] + parts[0][1]
    for p in parts[1:]:
        a = a + p[0] + p[1]
    z = (jnp.dot(h, w1_ref[:D], preferred_element_type=jnp.float32)
         + jnp.dot(a, w1_ref[D:], preferred_element_type=jnp.float32)
         + b1_ref[...])
    o = (jnp.dot(_silu(z), w2_ref[...], preferred_element_type=jnp.float32)
         + b2_ref[...])
    o_ref[...] = o + h if residual else o


def _node(h, parts, w1, b1, w2, b2, residual):
    return pl.pallas_call(
        functools.partial(_node_body, residual=residual),
        grid=(N // BN,),
        in_specs=[
            pl.BlockSpec((BN, D), lambda i: (i, 0)),
        ] + [
            pl.BlockSpec((NC, BN, D), lambda i: (0, i, 0))
        ] * NSEG + [
            pl.BlockSpec((2 * D, D), lambda i: (0, 0)),
            pl.BlockSpec((1, D), lambda i: (0, 0)),
            pl.BlockSpec((D, D), lambda i: (0, 0)),
            pl.BlockSpec((1, D), lambda i: (0, 0)),
        ],
        out_specs=pl.BlockSpec((BN, D), lambda i: (i, 0)),
        out_shape=jax.ShapeDtypeStruct((N, D), jnp.float32),
    )(h, *parts, w1, b1, w2, b2)


_MESH = plsc.VectorSubcoreMesh(core_axis_name="c", subcore_axis_name="s")


@functools.partial(
    pl.kernel,
    mesh=_MESH,
    out_type=(
        jax.ShapeDtypeStruct((ESEG, D), jnp.float32),
        jax.ShapeDtypeStruct((ESEG, D), jnp.float32),
        jax.ShapeDtypeStruct((ESEG,), jnp.float32),
    ),
    scratch_types=[
        pltpu.VMEM((EPW,), jnp.int32),
        pltpu.VMEM((EPW,), jnp.int32),
        pltpu.VMEM((EPW,), jnp.float32),
        pltpu.VMEM((N,), jnp.float32),
        pltpu.VMEM((N,), jnp.float32),
        pltpu.VMEM((N,), jnp.float32),
        pltpu.VMEM((ECHUNK, D), jnp.float32),
        pltpu.VMEM((ECHUNK, D), jnp.float32),
        pltpu.VMEM((ECHUNK, D), jnp.float32),
        pltpu.VMEM((ECHUNK, D), jnp.float32),
        pltpu.SemaphoreType.DMA,
        pltpu.SemaphoreType.DMA,
        pltpu.SemaphoreType.DMA,
        pltpu.SemaphoreType.DMA,
    ],
    compiler_params=pltpu.CompilerParams(needs_layout_passes=False),
)
def _gather(ts_hbm, td_hbm, xs_hbm, ys_hbm, zs_hbm, row_hbm, col_hbm,
            g1_hbm, g2_hbm, cr_hbm,
            ir_all, ic_all, cr_all, xs_v, ys_v, zs_v,
            b1a, b2a, b1b, b2b, sg0, sg1, sw0, sw1):
    wid = lax.axis_index("s") * NC + lax.axis_index("c")
    base = wid * EPW

    pltpu.sync_copy(row_hbm.at[pl.ds(base, EPW)], ir_all)
    pltpu.sync_copy(col_hbm.at[pl.ds(base, EPW)], ic_all)
    pltpu.sync_copy(xs_hbm, xs_v)
    pltpu.sync_copy(ys_hbm, ys_v)
    pltpu.sync_copy(zs_hbm, zs_v)

    def issue(c, b1, b2, sem):
        off = c * ECHUNK
        pltpu.async_copy(ts_hbm.at[ir_all.at[pl.ds(off, ECHUNK)]], b1, sem)
        pltpu.async_copy(td_hbm.at[ic_all.at[pl.ds(off, ECHUNK)]], b2, sem)

    def wait_g(b1, b2, sem):
        pltpu.make_async_copy(ts_hbm.at[pl.ds(0, ECHUNK)], b1, sem).wait()
        pltpu.make_async_copy(td_hbm.at[pl.ds(0, ECHUNK)], b2, sem).wait()

    def wb(c, b1, b2, sem):
        off = base + c * ECHUNK
        pltpu.async_copy(b1, g1_hbm.at[pl.ds(off, ECHUNK)], sem)
        pltpu.async_copy(b2, g2_hbm.at[pl.ds(off, ECHUNK)], sem)

    def wait_wb(b1, b2, sem):
        pltpu.make_async_copy(b1, g1_hbm.at[pl.ds(0, ECHUNK)], sem).wait()
        pltpu.make_async_copy(b2, g2_hbm.at[pl.ds(0, ECHUNK)], sem).wait()

    def cross(c):
        coff = c * ECHUNK
        for j in range(ECHUNK // 16):
            ii = ir_all[pl.ds(coff + j * 16, 16)]
            jj = ic_all[pl.ds(coff + j * 16, 16)]
            dot = (plsc.load_gather(xs_v, [ii]) * plsc.load_gather(xs_v, [jj])
                   + plsc.load_gather(ys_v, [ii]) * plsc.load_gather(ys_v, [jj])
                   + plsc.load_gather(zs_v, [ii]) * plsc.load_gather(zs_v, [jj]))
            cr_all[pl.ds(coff + j * 16, 16)] = -2.0 * dot

    issue(0, b1a, b2a, sg0)

    def body(j, carry):
        a = 2 * j
        b = a + 1

        @pl.when(j > 0)
        def _():
            wait_wb(b1b, b2b, sw1)
        issue(b, b1b, b2b, sg1)
        wait_g(b1a, b2a, sg0)
        cross(a)
        wb(a, b1a, b2a, sw0)
        wait_wb(b1a, b2a, sw0)
        issue(a + 2, b1a, b2a, sg0)
        wait_g(b1b, b2b, sg1)
        cross(b)
        wb(b, b1b, b2b, sw1)
        return carry

    lax.fori_loop(0, (NCHUNKS - 1) // 2, body, 0)

    last = NCHUNKS - 1
    wait_g(b1a, b2a, sg0)
    cross(last)
    wb(last, b1a, b2a, sw0)
    wait_wb(b1a, b2a, sw0)
    wait_wb(b1b, b2b, sw1)
    pltpu.sync_copy(cr_all, cr_hbm.at[pl.ds(base, EPW)])


@functools.partial(
    pl.kernel,
    mesh=_MESH,
    out_type=jax.ShapeDtypeStruct((NC, NP, D), jnp.float32),
    scratch_types=[
        pltpu.VMEM((ECHUNK,), jnp.int32),
        pltpu.VMEM((ECHUNK,), jnp.int32),
        pltpu.VMEM((ECHUNK, D), jnp.float32),
        pltpu.VMEM((ECHUNK, D), jnp.float32),
        pltpu.VMEM((ZROWS, D), jnp.float32),
        pltpu.VMEM_SHARED((NP, D), jnp.float32),
        pltpu.SemaphoreType.DMA,
        pltpu.SemaphoreType.DMA,
    ],
)
def _scatter(m_hbm, row_hbm, out_hbm, i0, i1, m0, m1, zb_v, acc_sh, s0, s1):
    cid = lax.axis_index("c")
    sid = lax.axis_index("s")
    wid = sid * NC + cid
    tbase = sid * NPT

    zero = jnp.zeros((16,), jnp.float32)

    def zrow(r, carry):
        def zcol(c, carry2):
            zb_v[r, pl.ds(c * 16, 16)] = zero
            return carry2
        return lax.fori_loop(0, D // 16, zcol, carry)

    lax.fori_loop(0, ZROWS, zrow, 0)

    def zcopy(j, carry):
        pltpu.sync_copy(zb_v, acc_sh.at[pl.ds(tbase + j * ZROWS, ZROWS)])
        return carry

    lax.fori_loop(0, NPT // ZROWS, zcopy, 0)
    plsc.subcore_barrier()

    base = wid * EPW

    def issue(c, iv, mv, sem):
        off = base + c * ECHUNK
        pltpu.async_copy(row_hbm.at[pl.ds(off, ECHUNK)], iv, sem)
        pltpu.async_copy(m_hbm.at[pl.ds(off, ECHUNK)], mv, sem)

    def wait_ld(iv, mv, sem):
        pltpu.make_async_copy(row_hbm.at[pl.ds(0, ECHUNK)], iv, sem).wait()
        pltpu.make_async_copy(m_hbm.at[pl.ds(0, ECHUNK)], mv, sem).wait()

    issue(0, i0, m0, s0)

    def body(j, carry):
        issue(2 * j + 1, i1, m1, s1)
        wait_ld(i0, m0, s0)
        pltpu.sync_copy(m0, acc_sh.at[i0], add=True)
        issue(2 * j + 2, i0, m0, s0)
        wait_ld(i1, m1, s1)
        pltpu.sync_copy(m1, acc_sh.at[i1], add=True)
        return carry

    lax.fori_loop(0, (NCHUNKS - 1) // 2, body, 0)
    wait_ld(i0, m0, s0)
    pltpu.sync_copy(m0, acc_sh.at[i0], add=True)
    plsc.subcore_barrier()

    pltpu.sync_copy(acc_sh.at[pl.ds(tbase, NPT)],
                    out_hbm.at[cid, pl.ds(tbase, NPT)])


def kernel(h, edges, coords, edge_attr, params):
    row, col = edges[0], edges[1]
    cpad = jnp.pad(coords, ((0, 0), (0, CD - 3)))
    xs, ys, zs = coords[:, 0], coords[:, 1], coords[:, 2]
    out = h
    for i, p in enumerate(params):
        w1 = p["edge_W1"]
        w1s, w1d = w1[:D], w1[D:2 * D]
        wr = w1[2 * D:2 * D + 1]
        w1e = w1[2 * D + 1:]
        b1 = p["edge_b1"][None, :]
        b2 = p["edge_b2"][None, :]
        ts, td = _tables(out, cpad, w1s, w1d, wr, b1)
        gs = [_gather(ts, td, xs, ys, zs,
                      row[s * ESEG:(s + 1) * ESEG],
                      col[s * ESEG:(s + 1) * ESEG]) for s in range(NSEG)]
        parts = []
        for s in range(NSEG):
            g1, g2, cross = gs[s]
            m = _edge(g1, g2, cross[:, None],
                      edge_attr[s * ESEG:(s + 1) * ESEG], w1e, wr,
                      p["edge_W2"], b2)
            parts.append(_scatter(m, row[s * ESEG:(s + 1) * ESEG]))
        out = _node(out, parts, p["node_W1"], p["node_b1"][None, :],
                    p["node_W2"], p["node_b2"][None, :], residual=(i > 0))
    return out


# speedup vs baseline: 3.9867x; 1.1560x over previous
import functools

import jax
import jax.numpy as jnp
from jax import lax
from jax.experimental import pallas as pl
from jax.experimental.pallas import tpu as pltpu
from jax.experimental.pallas import tpu_sc as plsc

N = 10000
E = 320000
D = 128
CD = 16
ED = 16

NC = 2
NS = 16
NW = NC * NS
NSEG = 5
ESEG = E // NSEG
EPW = ESEG // NW
ECHUNK = 80
NCHUNKS = EPW // ECHUNK
NP = 10240
NPT = NP // NS
ZROWS = 128

BN = 400
BE = 512


def _silu(x):
    return x * jax.nn.sigmoid(x)


def _tables_body(h_ref, c_ref, w1s_ref, w1d_ref, wr_ref, b1_ref,
                 ts_ref, td_ref):
    h = h_ref[...]
    c = c_ref[...]
    rw = jnp.sum(c * c, axis=1, keepdims=True) * wr_ref[...]
    ts_ref[...] = jnp.dot(h, w1s_ref[...], preferred_element_type=jnp.float32) + rw
    td_ref[...] = (jnp.dot(h, w1d_ref[...], preferred_element_type=jnp.float32)
                   + rw + b1_ref[...])


def _tables(h, cpad, w1s, w1d, wr, b1):
    return pl.pallas_call(
        _tables_body,
        grid=(N // BN,),
        in_specs=[
            pl.BlockSpec((BN, D), lambda i: (i, 0)),
            pl.BlockSpec((BN, CD), lambda i: (i, 0)),
            pl.BlockSpec((D, D), lambda i: (0, 0)),
            pl.BlockSpec((D, D), lambda i: (0, 0)),
            pl.BlockSpec((1, D), lambda i: (0, 0)),
            pl.BlockSpec((1, D), lambda i: (0, 0)),
        ],
        out_specs=[pl.BlockSpec((BN, D), lambda i: (i, 0))] * 2,
        out_shape=[jax.ShapeDtypeStruct((N, D), jnp.float32)] * 2,
    )(h, cpad, w1s, w1d, wr, b1)


def _edge_body(g1_ref, g2_ref, cr_ref, ea_ref, w1e_ref, wr_ref,
               w2_ref, b2_ref, m_ref):
    cr = cr_ref[...].reshape(BE, 1)
    pre = (g1_ref[...] + g2_ref[...]
           + jnp.dot(ea_ref[...], w1e_ref[...], preferred_element_type=jnp.float32)
           + cr * wr_ref[...])
    m_ref[...] = _silu(
        jnp.dot(_silu(pre), w2_ref[...], preferred_element_type=jnp.float32)
        + b2_ref[...])


def _edge(g1, g2, cross, ea, w1e, wr, w2, b2):
    return pl.pallas_call(
        _edge_body,
        grid=(ESEG // BE,),
        in_specs=[
            pl.BlockSpec((BE, D), lambda i: (i, 0)),
            pl.BlockSpec((BE, D), lambda i: (i, 0)),
            pl.BlockSpec((1, 1, BE), lambda i: (i, 0, 0)),
            pl.BlockSpec((BE, ED), lambda i: (i, 0)),
            pl.BlockSpec((ED, D), lambda i: (0, 0)),
            pl.BlockSpec((1, D), lambda i: (0, 0)),
            pl.BlockSpec((D, D), lambda i: (0, 0)),
            pl.BlockSpec((1, D), lambda i: (0, 0)),
        ],
        out_specs=pl.BlockSpec((BE, D), lambda i: (i, 0)),
        out_shape=jax.ShapeDtypeStruct((ESEG, D), jnp.float32),
    )(g1, g2, cross, ea, w1e, wr, w2, b2)


def _node_body(h_ref, *rest, residual):
    parts = rest[:NSEG]
    w1_ref, b1_ref, w2_ref, b2_ref, o_ref = rest[NSEG:]
    h = h_ref[...]
    a = parts[0][0] + parts[0][1]
    for p in parts[1:]:
        a = a + p[0] + p[1]
    z = (jnp.dot(h, w1_ref[:D], preferred_element_type=jnp.float32)
         + jnp.dot(a, w1_ref[D:], preferred_element_type=jnp.float32)
         + b1_ref[...])
    o = (jnp.dot(_silu(z), w2_ref[...], preferred_element_type=jnp.float32)
         + b2_ref[...])
    o_ref[...] = o + h if residual else o


def _node(h, parts, w1, b1, w2, b2, residual):
    return pl.pallas_call(
        functools.partial(_node_body, residual=residual),
        grid=(N // BN,),
        in_specs=[
            pl.BlockSpec((BN, D), lambda i: (i, 0)),
        ] + [
            pl.BlockSpec((NC, BN, D), lambda i: (0, i, 0))
        ] * NSEG + [
            pl.BlockSpec((2 * D, D), lambda i: (0, 0)),
            pl.BlockSpec((1, D), lambda i: (0, 0)),
            pl.BlockSpec((D, D), lambda i: (0, 0)),
            pl.BlockSpec((1, D), lambda i: (0, 0)),
        ],
        out_specs=pl.BlockSpec((BN, D), lambda i: (i, 0)),
        out_shape=jax.ShapeDtypeStruct((N, D), jnp.float32),
    )(h, *parts, w1, b1, w2, b2)


_MESH = plsc.VectorSubcoreMesh(core_axis_name="c", subcore_axis_name="s")


@functools.partial(
    pl.kernel,
    mesh=_MESH,
    out_type=(
        jax.ShapeDtypeStruct((ESEG, D), jnp.float32),
        jax.ShapeDtypeStruct((ESEG, D), jnp.float32),
        jax.ShapeDtypeStruct((ESEG,), jnp.float32),
    ),
    scratch_types=[
        pltpu.VMEM((EPW,), jnp.int32),
        pltpu.VMEM((EPW,), jnp.int32),
        pltpu.VMEM((EPW,), jnp.float32),
        pltpu.VMEM((N,), jnp.float32),
        pltpu.VMEM((N,), jnp.float32),
        pltpu.VMEM((N,), jnp.float32),
        pltpu.VMEM((ECHUNK, D), jnp.float32),
        pltpu.VMEM((ECHUNK, D), jnp.float32),
        pltpu.VMEM((ECHUNK, D), jnp.float32),
        pltpu.VMEM((ECHUNK, D), jnp.float32),
        pltpu.SemaphoreType.DMA,
        pltpu.SemaphoreType.DMA,
        pltpu.SemaphoreType.DMA,
        pltpu.SemaphoreType.DMA,
    ],
    compiler_params=pltpu.CompilerParams(needs_layout_passes=False),
)
def _gather(ts_hbm, td_hbm, xs_hbm, ys_hbm, zs_hbm, row_hbm, col_hbm,
            g1_hbm, g2_hbm, cr_hbm,
            ir_all, ic_all, cr_all, xs_v, ys_v, zs_v,
            b1a, b2a, b1b, b2b, sg0, sg1, sw0, sw1):
    wid = lax.axis_index("s") * NC + lax.axis_index("c")
    base = wid * EPW

    pltpu.sync_copy(row_hbm.at[pl.ds(base, EPW)], ir_all)
    pltpu.sync_copy(col_hbm.at[pl.ds(base, EPW)], ic_all)
    pltpu.sync_copy(xs_hbm, xs_v)
    pltpu.sync_copy(ys_hbm, ys_v)
    pltpu.sync_copy(zs_hbm, zs_v)

    def issue(c, b1, b2, sem):
        off = c * ECHUNK
        pltpu.async_copy(ts_hbm.at[ir_all.at[pl.ds(off, ECHUNK)]], b1, sem)
        pltpu.async_copy(td_hbm.at[ic_all.at[pl.ds(off, ECHUNK)]], b2, sem)

    def wait_g(b1, b2, sem):
        pltpu.make_async_copy(ts_hbm.at[pl.ds(0, ECHUNK)], b1, sem).wait()
        pltpu.make_async_copy(td_hbm.at[pl.ds(0, ECHUNK)], b2, sem).wait()

    def wb(c, b1, b2, sem):
        off = base + c * ECHUNK
        pltpu.async_copy(b1, g1_hbm.at[pl.ds(off, ECHUNK)], sem)
        pltpu.async_copy(b2, g2_hbm.at[pl.ds(off, ECHUNK)], sem)

    def wait_wb(b1, b2, sem):
        pltpu.make_async_copy(b1, g1_hbm.at[pl.ds(0, ECHUNK)], sem).wait()
        pltpu.make_async_copy(b2, g2_hbm.at[pl.ds(0, ECHUNK)], sem).wait()

    def cross(c):
        coff = c * ECHUNK
        for j in range(ECHUNK // 16):
            ii = ir_all[pl.ds(coff + j * 16, 16)]
            jj = ic_all[pl.ds(coff + j * 16, 16)]
            dot = (plsc.load_gather(xs_v, [ii]) * plsc.load_gather(xs_v, [jj])
                   + plsc.load_gather(ys_v, [ii]) * plsc.load_gather(ys_v, [jj])
                   + plsc.load_gather(zs_v, [ii]) * plsc.load_gather(zs_v, [jj]))
            cr_all[pl.ds(coff + j * 16, 16)] = -2.0 * dot

    issue(0, b1a, b2a, sg0)

    def body(j, carry):
        a = 2 * j
        b = a + 1

        @pl.when(j > 0)
        def _():
            wait_wb(b1b, b2b, sw1)
        issue(b, b1b, b2b, sg1)
        wait_g(b1a, b2a, sg0)
        cross(a)
        wb(a, b1a, b2a, sw0)
        wait_wb(b1a, b2a, sw0)
        issue(a + 2, b1a, b2a, sg0)
        wait_g(b1b, b2b, sg1)
        cross(b)
        wb(b, b1b, b2b, sw1)
        return carry

    lax.fori_loop(0, (NCHUNKS - 1) // 2, body, 0)

    last = NCHUNKS - 1
    wait_g(b1a, b2a, sg0)
    cross(last)
    wb(last, b1a, b2a, sw0)
    wait_wb(b1a, b2a, sw0)
    wait_wb(b1b, b2b, sw1)
    pltpu.sync_copy(cr_all, cr_hbm.at[pl.ds(base, EPW)])


@functools.partial(
    pl.kernel,
    mesh=_MESH,
    out_type=jax.ShapeDtypeStruct((NC, NP, D), jnp.float32),
    scratch_types=[
        pltpu.VMEM((ECHUNK,), jnp.int32),
        pltpu.VMEM((ECHUNK,), jnp.int32),
        pltpu.VMEM((ECHUNK, D), jnp.float32),
        pltpu.VMEM((ECHUNK, D), jnp.float32),
        pltpu.VMEM((ZROWS, D), jnp.float32),
        pltpu.VMEM_SHARED((NP, D), jnp.float32),
        pltpu.SemaphoreType.DMA,
        pltpu.SemaphoreType.DMA,
    ],
)
def _scatter(m_hbm, row_hbm, out_hbm, i0, i1, m0, m1, zb_v, acc_sh, s0, s1):
    cid = lax.axis_index("c")
    sid = lax.axis_index("s")
    wid = sid * NC + cid
    tbase = sid * NPT

    zero = jnp.zeros((16,), jnp.float32)

    def zrow(r, carry):
        def zcol(c, carry2):
            zb_v[r, pl.ds(c * 16, 16)] = zero
            return carry2
        return lax.fori_loop(0, D // 16, zcol, carry)

    lax.fori_loop(0, ZROWS, zrow, 0)

    def zcopy(j, carry):
        pltpu.sync_copy(zb_v, acc_sh.at[pl.ds(tbase + j * ZROWS, ZROWS)])
        return carry

    lax.fori_loop(0, NPT // ZROWS, zcopy, 0)
    plsc.subcore_barrier()

    base = wid * EPW

    def issue(c, iv, mv, sem):
        off = base + c * ECHUNK
        pltpu.async_copy(row_hbm.at[pl.ds(off, ECHUNK)], iv, sem)
        pltpu.async_copy(m_hbm.at[pl.ds(off, ECHUNK)], mv, sem)

    def wait_ld(iv, mv, sem):
        pltpu.make_async_copy(row_hbm.at[pl.ds(0, ECHUNK)], iv, sem).wait()
        pltpu.make_async_copy(m_hbm.at[pl.ds(0, ECHUNK)], mv, sem).wait()

    issue(0, i0, m0, s0)

    def body(j, carry):
        issue(2 * j + 1, i1, m1, s1)
        wait_ld(i0, m0, s0)
        pltpu.sync_copy(m0, acc_sh.at[i0], add=True)
        issue(2 * j + 2, i0, m0, s0)
        wait_ld(i1, m1, s1)
        pltpu.sync_copy(m1, acc_sh.at[i1], add=True)
        return carry

    lax.fori_loop(0, (NCHUNKS - 1) // 2, body, 0)
    wait_ld(i0, m0, s0)
    pltpu.sync_copy(m0, acc_sh.at[i0], add=True)
    plsc.subcore_barrier()

    pltpu.sync_copy(acc_sh.at[pl.ds(tbase, NPT)],
                    out_hbm.at[cid, pl.ds(tbase, NPT)])


def kernel(h, edges, coords, edge_attr, params):
    row, col = edges[0], edges[1]
    cpad = jnp.pad(coords, ((0, 0), (0, CD - 3)))
    xs, ys, zs = coords[:, 0], coords[:, 1], coords[:, 2]
    out = h
    for i, p in enumerate(params):
        w1 = p["edge_W1"]
        w1s, w1d = w1[:D], w1[D:2 * D]
        wr = w1[2 * D:2 * D + 1]
        w1e = w1[2 * D + 1:]
        b1 = p["edge_b1"][None, :]
        b2 = p["edge_b2"][None, :]
        ts, td = _tables(out, cpad, w1s, w1d, wr, b1)
        gs = [_gather(ts, td, xs, ys, zs,
                      row[s * ESEG:(s + 1) * ESEG],
                      col[s * ESEG:(s + 1) * ESEG]) for s in range(NSEG)]
        parts = []
        for s in range(NSEG):
            g1, g2, cross = gs[s]
            m = _edge(g1, g2, cross.reshape(ESEG // BE, 1, BE),
                      edge_attr[s * ESEG:(s + 1) * ESEG], w1e, wr,
                      p["edge_W2"], b2)
            parts.append(_scatter(m, row[s * ESEG:(s + 1) * ESEG]))
        out = _node(out, parts, p["node_W1"], p["node_b1"][None, :],
                    p["node_W2"], p["node_b2"][None, :], residual=(i > 0))
    return out


# speedup vs baseline: 4.3139x; 1.0821x over previous
import functools

import jax
import jax.numpy as jnp
from jax import lax
from jax.experimental import pallas as pl
from jax.experimental.pallas import tpu as pltpu
from jax.experimental.pallas import tpu_sc as plsc

N = 10000
E = 320000
D = 128
CD = 16
ED = 16

NC = 2
NS = 16
NW = NC * NS
NSEG = 5
ESEG = E // NSEG
EPW = ESEG // NW
ECHUNK = 80
NCHUNKS = EPW // ECHUNK
NP = 10240
NPT = NP // NS
ZROWS = 128

BN = 400
BE = 640


def _silu(x):
    return x * jax.nn.sigmoid(x)


def _tables_body(h_ref, c_ref, w1s_ref, w1d_ref, wr_ref, b1_ref,
                 ts_ref, td_ref):
    h = h_ref[...]
    c = c_ref[...]
    rw = jnp.sum(c * c, axis=1, keepdims=True) * wr_ref[...]
    ts_ref[...] = jnp.dot(h, w1s_ref[...], preferred_element_type=jnp.float32) + rw
    td_ref[...] = (jnp.dot(h, w1d_ref[...], preferred_element_type=jnp.float32)
                   + rw + b1_ref[...])


def _tables(h, cpad, w1s, w1d, wr, b1):
    return pl.pallas_call(
        _tables_body,
        grid=(N // BN,),
        in_specs=[
            pl.BlockSpec((BN, D), lambda i: (i, 0)),
            pl.BlockSpec((BN, CD), lambda i: (i, 0)),
            pl.BlockSpec((D, D), lambda i: (0, 0)),
            pl.BlockSpec((D, D), lambda i: (0, 0)),
            pl.BlockSpec((1, D), lambda i: (0, 0)),
            pl.BlockSpec((1, D), lambda i: (0, 0)),
        ],
        out_specs=[pl.BlockSpec((BN, D), lambda i: (i, 0))] * 2,
        out_shape=[jax.ShapeDtypeStruct((N, D), jnp.float32)] * 2,
    )(h, cpad, w1s, w1d, wr, b1)


def _edge_body(g_ref, cr_ref, ea_ref, w1e_ref, wr_ref,
               w2_ref, b2_ref, m_ref):
    cr = cr_ref[...].reshape(BE, 1)
    pre = (g_ref[0] + g_ref[1]
           + jnp.dot(ea_ref[...], w1e_ref[...], preferred_element_type=jnp.float32)
           + cr * wr_ref[...])
    m_ref[...] = _silu(
        jnp.dot(_silu(pre), w2_ref[...], preferred_element_type=jnp.float32)
        + b2_ref[...])


def _edge(g, cross, ea, w1e, wr, w2, b2):
    return pl.pallas_call(
        _edge_body,
        grid=(ESEG // BE,),
        in_specs=[
            pl.BlockSpec((2, BE, D), lambda i: (0, i, 0)),
            pl.BlockSpec((1, 1, BE), lambda i: (i, 0, 0)),
            pl.BlockSpec((BE, ED), lambda i: (i, 0)),
            pl.BlockSpec((ED, D), lambda i: (0, 0)),
            pl.BlockSpec((1, D), lambda i: (0, 0)),
            pl.BlockSpec((D, D), lambda i: (0, 0)),
            pl.BlockSpec((1, D), lambda i: (0, 0)),
        ],
        out_specs=pl.BlockSpec((BE, D), lambda i: (i, 0)),
        out_shape=jax.ShapeDtypeStruct((ESEG, D), jnp.float32),
    )(g, cross, ea, w1e, wr, w2, b2)


def _node_body(h_ref, *rest, residual):
    parts = rest[:NSEG]
    w1_ref, b1_ref, w2_ref, b2_ref, o_ref = rest[NSEG:]
    h = h_ref[...]
    a = parts[0][0] + parts[0][1]
    for p in parts[1:]:
        a = a + p[0] + p[1]
    z = (jnp.dot(h, w1_ref[:D], preferred_element_type=jnp.float32)
         + jnp.dot(a, w1_ref[D:], preferred_element_type=jnp.float32)
         + b1_ref[...])
    o = (jnp.dot(_silu(z), w2_ref[...], preferred_element_type=jnp.float32)
         + b2_ref[...])
    o_ref[...] = o + h if residual else o


def _node(h, parts, w1, b1, w2, b2, residual):
    return pl.pallas_call(
        functools.partial(_node_body, residual=residual),
        grid=(N // BN,),
        in_specs=[
            pl.BlockSpec((BN, D), lambda i: (i, 0)),
        ] + [
            pl.BlockSpec((NC, BN, D), lambda i: (0, i, 0))
        ] * NSEG + [
            pl.BlockSpec((2 * D, D), lambda i: (0, 0)),
            pl.BlockSpec((1, D), lambda i: (0, 0)),
            pl.BlockSpec((D, D), lambda i: (0, 0)),
            pl.BlockSpec((1, D), lambda i: (0, 0)),
        ],
        out_specs=pl.BlockSpec((BN, D), lambda i: (i, 0)),
        out_shape=jax.ShapeDtypeStruct((N, D), jnp.float32),
    )(h, *parts, w1, b1, w2, b2)


_MESH = plsc.VectorSubcoreMesh(core_axis_name="c", subcore_axis_name="s")


@functools.partial(
    pl.kernel,
    mesh=_MESH,
    out_type=(
        jax.ShapeDtypeStruct((2, ESEG, D), jnp.float32),
        jax.ShapeDtypeStruct((ESEG,), jnp.float32),
    ),
    scratch_types=[
        pltpu.VMEM((EPW,), jnp.int32),
        pltpu.VMEM((EPW,), jnp.int32),
        pltpu.VMEM((EPW,), jnp.float32),
        pltpu.VMEM((N,), jnp.float32),
        pltpu.VMEM((N,), jnp.float32),
        pltpu.VMEM((N,), jnp.float32),
        pltpu.VMEM((ECHUNK, D), jnp.float32),
        pltpu.VMEM((ECHUNK, D), jnp.float32),
        pltpu.VMEM((ECHUNK, D), jnp.float32),
        pltpu.VMEM((ECHUNK, D), jnp.float32),
        pltpu.SemaphoreType.DMA,
        pltpu.SemaphoreType.DMA,
        pltpu.SemaphoreType.DMA,
        pltpu.SemaphoreType.DMA,
    ],
    compiler_params=pltpu.CompilerParams(needs_layout_passes=False),
)
def _gather(ts_hbm, td_hbm, xs_hbm, ys_hbm, zs_hbm, row_hbm, col_hbm,
            g_hbm, cr_hbm,
            ir_all, ic_all, cr_all, xs_v, ys_v, zs_v,
            b1a, b2a, b1b, b2b, sg0, sg1, sw0, sw1):
    wid = lax.axis_index("s") * NC + lax.axis_index("c")
    base = wid * EPW

    pltpu.sync_copy(row_hbm.at[pl.ds(base, EPW)], ir_all)
    pltpu.sync_copy(col_hbm.at[pl.ds(base, EPW)], ic_all)
    pltpu.sync_copy(xs_hbm, xs_v)
    pltpu.sync_copy(ys_hbm, ys_v)
    pltpu.sync_copy(zs_hbm, zs_v)

    def issue(c, b1, b2, sem):
        off = c * ECHUNK
        pltpu.async_copy(ts_hbm.at[ir_all.at[pl.ds(off, ECHUNK)]], b1, sem)
        pltpu.async_copy(td_hbm.at[ic_all.at[pl.ds(off, ECHUNK)]], b2, sem)

    def wait_g(b1, b2, sem):
        pltpu.make_async_copy(ts_hbm.at[pl.ds(0, ECHUNK)], b1, sem).wait()
        pltpu.make_async_copy(td_hbm.at[pl.ds(0, ECHUNK)], b2, sem).wait()

    def wb(c, b1, b2, sem):
        off = base + c * ECHUNK
        pltpu.async_copy(b1, g_hbm.at[0, pl.ds(off, ECHUNK)], sem)
        pltpu.async_copy(b2, g_hbm.at[1, pl.ds(off, ECHUNK)], sem)

    def wait_wb(b1, b2, sem):
        pltpu.make_async_copy(b1, g_hbm.at[0, pl.ds(0, ECHUNK)], sem).wait()
        pltpu.make_async_copy(b2, g_hbm.at[1, pl.ds(0, ECHUNK)], sem).wait()

    def cross(c):
        coff = c * ECHUNK
        for j in range(ECHUNK // 16):
            ii = ir_all[pl.ds(coff + j * 16, 16)]
            jj = ic_all[pl.ds(coff + j * 16, 16)]
            dot = (plsc.load_gather(xs_v, [ii]) * plsc.load_gather(xs_v, [jj])
                   + plsc.load_gather(ys_v, [ii]) * plsc.load_gather(ys_v, [jj])
                   + plsc.load_gather(zs_v, [ii]) * plsc.load_gather(zs_v, [jj]))
            cr_all[pl.ds(coff + j * 16, 16)] = -2.0 * dot

    issue(0, b1a, b2a, sg0)

    def body(j, carry):
        a = 2 * j
        b = a + 1

        @pl.when(j > 0)
        def _():
            wait_wb(b1b, b2b, sw1)
        issue(b, b1b, b2b, sg1)
        wait_g(b1a, b2a, sg0)
        cross(a)
        wb(a, b1a, b2a, sw0)
        wait_wb(b1a, b2a, sw0)
        issue(a + 2, b1a, b2a, sg0)
        wait_g(b1b, b2b, sg1)
        cross(b)
        wb(b, b1b, b2b, sw1)
        return carry

    lax.fori_loop(0, (NCHUNKS - 1) // 2, body, 0)

    last = NCHUNKS - 1
    wait_g(b1a, b2a, sg0)
    cross(last)
    wb(last, b1a, b2a, sw0)
    wait_wb(b1a, b2a, sw0)
    wait_wb(b1b, b2b, sw1)
    pltpu.sync_copy(cr_all, cr_hbm.at[pl.ds(base, EPW)])


@functools.partial(
    pl.kernel,
    mesh=_MESH,
    out_type=jax.ShapeDtypeStruct((NC, NP, D), jnp.float32),
    scratch_types=[
        pltpu.VMEM((ECHUNK,), jnp.int32),
        pltpu.VMEM((ECHUNK,), jnp.int32),
        pltpu.VMEM((ECHUNK, D), jnp.float32),
        pltpu.VMEM((ECHUNK, D), jnp.float32),
        pltpu.VMEM((ZROWS, D), jnp.float32),
        pltpu.VMEM_SHARED((NP, D), jnp.float32),
        pltpu.SemaphoreType.DMA,
        pltpu.SemaphoreType.DMA,
    ],
)
def _scatter(m_hbm, row_hbm, out_hbm, i0, i1, m0, m1, zb_v, acc_sh, s0, s1):
    cid = lax.axis_index("c")
    sid = lax.axis_index("s")
    wid = sid * NC + cid
    tbase = sid * NPT

    zero = jnp.zeros((16,), jnp.float32)

    def zrow(r, carry):
        def zcol(c, carry2):
            zb_v[r, pl.ds(c * 16, 16)] = zero
            return carry2
        return lax.fori_loop(0, D // 16, zcol, carry)

    lax.fori_loop(0, ZROWS, zrow, 0)

    def zcopy(j, carry):
        pltpu.sync_copy(zb_v, acc_sh.at[pl.ds(tbase + j * ZROWS, ZROWS)])
        return carry

    lax.fori_loop(0, NPT // ZROWS, zcopy, 0)
    plsc.subcore_barrier()

    base = wid * EPW

    def issue(c, iv, mv, sem):
        off = base + c * ECHUNK
        pltpu.async_copy(row_hbm.at[pl.ds(off, ECHUNK)], iv, sem)
        pltpu.async_copy(m_hbm.at[pl.ds(off, ECHUNK)], mv, sem)

    def wait_ld(iv, mv, sem):
        pltpu.make_async_copy(row_hbm.at[pl.ds(0, ECHUNK)], iv, sem).wait()
        pltpu.make_async_copy(m_hbm.at[pl.ds(0, ECHUNK)], mv, sem).wait()

    issue(0, i0, m0, s0)

    def body(j, carry):
        issue(2 * j + 1, i1, m1, s1)
        wait_ld(i0, m0, s0)
        pltpu.sync_copy(m0, acc_sh.at[i0], add=True)
        issue(2 * j + 2, i0, m0, s0)
        wait_ld(i1, m1, s1)
        pltpu.sync_copy(m1, acc_sh.at[i1], add=True)
        return carry

    lax.fori_loop(0, (NCHUNKS - 1) // 2, body, 0)
    wait_ld(i0, m0, s0)
    pltpu.sync_copy(m0, acc_sh.at[i0], add=True)
    plsc.subcore_barrier()

    pltpu.sync_copy(acc_sh.at[pl.ds(tbase, NPT)],
                    out_hbm.at[cid, pl.ds(tbase, NPT)])


def kernel(h, edges, coords, edge_attr, params):
    row, col = edges[0], edges[1]
    cpad = jnp.pad(coords, ((0, 0), (0, CD - 3)))
    xs, ys, zs = coords[:, 0], coords[:, 1], coords[:, 2]
    out = h
    for i, p in enumerate(params):
        w1 = p["edge_W1"]
        w1s, w1d = w1[:D], w1[D:2 * D]
        wr = w1[2 * D:2 * D + 1]
        w1e = w1[2 * D + 1:]
        b1 = p["edge_b1"][None, :]
        b2 = p["edge_b2"][None, :]
        ts, td = _tables(out, cpad, w1s, w1d, wr, b1)
        gs = [_gather(ts, td, xs, ys, zs,
                      row[s * ESEG:(s + 1) * ESEG],
                      col[s * ESEG:(s + 1) * ESEG]) for s in range(NSEG)]
        parts = []
        for s in range(NSEG):
            g, cross = gs[s]
            m = _edge(g, cross.reshape(ESEG // BE, 1, BE),
                      edge_attr[s * ESEG:(s + 1) * ESEG], w1e, wr,
                      p["edge_W2"], b2)
            parts.append(_scatter(m, row[s * ESEG:(s + 1) * ESEG]))
        out = _node(out, parts, p["node_W1"], p["node_b1"][None, :],
                    p["node_W2"], p["node_b2"][None, :], residual=(i > 0))
    return out


# speedup vs baseline: 4.3175x; 1.0008x over previous
import functools

import jax
import jax.numpy as jnp
from jax import lax
from jax.experimental import pallas as pl
from jax.experimental.pallas import tpu as pltpu
from jax.experimental.pallas import tpu_sc as plsc

N = 10000
E = 320000
D = 128
CD = 16
ED = 16

NC = 2
NS = 16
NW = NC * NS
NSEG = 5
ESEG = E // NSEG
EPW = ESEG // NW
ECHUNK = 80
NCHUNKS = EPW // ECHUNK
NP = 10240
NPT = NP // NS
ZROWS = 128

BN = 400
BE = 640


def _silu(x):
    return x * jax.nn.sigmoid(x)


def _tables_body(h_ref, c_ref, w1s_ref, w1d_ref, wr_ref, b1_ref,
                 ts_ref, td_ref):
    h = h_ref[...]
    c = c_ref[...]
    rw = jnp.sum(c * c, axis=1, keepdims=True) * wr_ref[...]
    ts_ref[...] = jnp.dot(h, w1s_ref[...], preferred_element_type=jnp.float32) + rw
    td_ref[...] = (jnp.dot(h, w1d_ref[...], preferred_element_type=jnp.float32)
                   + rw + b1_ref[...])


def _tables(h, cpad, w1s, w1d, wr, b1):
    return pl.pallas_call(
        _tables_body,
        grid=(N // BN,),
        in_specs=[
            pl.BlockSpec((BN, D), lambda i: (i, 0)),
            pl.BlockSpec((BN, CD), lambda i: (i, 0)),
            pl.BlockSpec((D, D), lambda i: (0, 0)),
            pl.BlockSpec((D, D), lambda i: (0, 0)),
            pl.BlockSpec((1, D), lambda i: (0, 0)),
            pl.BlockSpec((1, D), lambda i: (0, 0)),
        ],
        out_specs=[pl.BlockSpec((BN, D), lambda i: (i, 0))] * 2,
        out_shape=[jax.ShapeDtypeStruct((N, D), jnp.float32)] * 2,
    )(h, cpad, w1s, w1d, wr, b1)


def _edge_body(g_ref, cr_ref, ea_ref, w1e_ref, wr_ref,
               w2_ref, b2_ref, m_ref):
    cr = cr_ref[...].reshape(BE, 1)
    pre = (g_ref[0] + g_ref[1]
           + jnp.dot(ea_ref[...], w1e_ref[...], preferred_element_type=jnp.float32)
           + cr * wr_ref[...])
    m = _silu(
        jnp.dot(_silu(pre), w2_ref[...], preferred_element_type=jnp.float32)
        + b2_ref[...])
    m_ref[...] = m


def _edge(g, cross, ea, w1e, wr, w2, b2):
    return pl.pallas_call(
        _edge_body,
        grid=(ESEG // BE,),
        in_specs=[
            pl.BlockSpec((2, BE, D), lambda i: (0, i, 0)),
            pl.BlockSpec((1, 1, BE), lambda i: (i, 0, 0)),
            pl.BlockSpec((BE, ED), lambda i: (i, 0)),
            pl.BlockSpec((ED, D), lambda i: (0, 0)),
            pl.BlockSpec((1, D), lambda i: (0, 0)),
            pl.BlockSpec((D, D), lambda i: (0, 0)),
            pl.BlockSpec((1, D), lambda i: (0, 0)),
        ],
        out_specs=pl.BlockSpec((BE, D), lambda i: (i, 0)),
        out_shape=jax.ShapeDtypeStruct((ESEG, D), jnp.float32),
    )(g, cross, ea, w1e, wr, w2, b2)


def _node_body(h_ref, *rest, residual):
    parts = rest[:NSEG]
    w1_ref, b1_ref, w2_ref, b2_ref, o_ref = rest[NSEG:]
    h = h_ref[...]
    a = parts[0][0] + parts[0][1]
    for p in parts[1:]:
        a = a + p[0] + p[1]
    z = (jnp.dot(h, w1_ref[:D], preferred_element_type=jnp.float32)
         + jnp.dot(a, w1_ref[D:], preferred_element_type=jnp.float32)
         + b1_ref[...])
    o = (jnp.dot(_silu(z), w2_ref[...], preferred_element_type=jnp.float32)
         + b2_ref[...])
    o_ref[...] = o + h if residual else o


def _node(h, parts, w1, b1, w2, b2, residual):
    return pl.pallas_call(
        functools.partial(_node_body, residual=residual),
        grid=(N // BN,),
        in_specs=[
            pl.BlockSpec((BN, D), lambda i: (i, 0)),
        ] + [
            pl.BlockSpec((NC, BN, D), lambda i: (0, i, 0))
        ] * NSEG + [
            pl.BlockSpec((2 * D, D), lambda i: (0, 0)),
            pl.BlockSpec((1, D), lambda i: (0, 0)),
            pl.BlockSpec((D, D), lambda i: (0, 0)),
            pl.BlockSpec((1, D), lambda i: (0, 0)),
        ],
        out_specs=pl.BlockSpec((BN, D), lambda i: (i, 0)),
        out_shape=jax.ShapeDtypeStruct((N, D), jnp.float32),
    )(h, *parts, w1, b1, w2, b2)


_MESH = plsc.VectorSubcoreMesh(core_axis_name="c", subcore_axis_name="s")


@functools.partial(
    pl.kernel,
    mesh=_MESH,
    out_type=(
        jax.ShapeDtypeStruct((2, ESEG, D), jnp.float32),
        jax.ShapeDtypeStruct((ESEG,), jnp.float32),
    ),
    scratch_types=[
        pltpu.VMEM((EPW,), jnp.int32),
        pltpu.VMEM((EPW,), jnp.int32),
        pltpu.VMEM((EPW,), jnp.float32),
        pltpu.VMEM((N,), jnp.float32),
        pltpu.VMEM((N,), jnp.float32),
        pltpu.VMEM((N,), jnp.float32),
        pltpu.VMEM((ECHUNK, D), jnp.float32),
        pltpu.VMEM((ECHUNK, D), jnp.float32),
        pltpu.VMEM((ECHUNK, D), jnp.float32),
        pltpu.VMEM((ECHUNK, D), jnp.float32),
        pltpu.SemaphoreType.DMA,
        pltpu.SemaphoreType.DMA,
        pltpu.SemaphoreType.DMA,
        pltpu.SemaphoreType.DMA,
    ],
    compiler_params=pltpu.CompilerParams(needs_layout_passes=False),
)
def _gather(ts_hbm, td_hbm, xs_hbm, ys_hbm, zs_hbm, row_hbm, col_hbm,
            g_hbm, cr_hbm,
            ir_all, ic_all, cr_all, xs_v, ys_v, zs_v,
            b1a, b2a, b1b, b2b, sg0, sg1, sw0, sw1):
    wid = lax.axis_index("s") * NC + lax.axis_index("c")
    base = wid * EPW

    pltpu.sync_copy(row_hbm.at[pl.ds(base, EPW)], ir_all)
    pltpu.sync_copy(col_hbm.at[pl.ds(base, EPW)], ic_all)
    pltpu.sync_copy(xs_hbm, xs_v)
    pltpu.sync_copy(ys_hbm, ys_v)
    pltpu.sync_copy(zs_hbm, zs_v)

    def issue(c, b1, b2, sem):
        off = c * ECHUNK
        pltpu.async_copy(ts_hbm.at[ir_all.at[pl.ds(off, ECHUNK)]], b1, sem)
        pltpu.async_copy(td_hbm.at[ic_all.at[pl.ds(off, ECHUNK)]], b2, sem)

    def wait_g(b1, b2, sem):
        pltpu.make_async_copy(ts_hbm.at[pl.ds(0, ECHUNK)], b1, sem).wait()
        pltpu.make_async_copy(td_hbm.at[pl.ds(0, ECHUNK)], b2, sem).wait()

    def wb(c, b1, b2, sem):
        off = base + c * ECHUNK
        pltpu.async_copy(b1, g_hbm.at[0, pl.ds(off, ECHUNK)], sem)
        pltpu.async_copy(b2, g_hbm.at[1, pl.ds(off, ECHUNK)], sem)

    def wait_wb(b1, b2, sem):
        pltpu.make_async_copy(b1, g_hbm.at[0, pl.ds(0, ECHUNK)], sem).wait()
        pltpu.make_async_copy(b2, g_hbm.at[1, pl.ds(0, ECHUNK)], sem).wait()

    def cross(c):
        coff = c * ECHUNK
        for j in range(ECHUNK // 16):
            ii = ir_all[pl.ds(coff + j * 16, 16)]
            jj = ic_all[pl.ds(coff + j * 16, 16)]
            dot = (plsc.load_gather(xs_v, [ii]) * plsc.load_gather(xs_v, [jj])
                   + plsc.load_gather(ys_v, [ii]) * plsc.load_gather(ys_v, [jj])
                   + plsc.load_gather(zs_v, [ii]) * plsc.load_gather(zs_v, [jj]))
            cr_all[pl.ds(coff + j * 16, 16)] = -2.0 * dot

    issue(0, b1a, b2a, sg0)

    def body(j, carry):
        a = 2 * j
        b = a + 1

        @pl.when(j > 0)
        def _():
            wait_wb(b1b, b2b, sw1)
        issue(b, b1b, b2b, sg1)
        wait_g(b1a, b2a, sg0)
        cross(a)
        wb(a, b1a, b2a, sw0)
        wait_wb(b1a, b2a, sw0)
        issue(a + 2, b1a, b2a, sg0)
        wait_g(b1b, b2b, sg1)
        cross(b)
        wb(b, b1b, b2b, sw1)
        return carry

    lax.fori_loop(0, (NCHUNKS - 1) // 2, body, 0)

    last = NCHUNKS - 1
    wait_g(b1a, b2a, sg0)
    cross(last)
    wb(last, b1a, b2a, sw0)
    wait_wb(b1a, b2a, sw0)
    wait_wb(b1b, b2b, sw1)
    pltpu.sync_copy(cr_all, cr_hbm.at[pl.ds(base, EPW)])


@functools.partial(
    pl.kernel,
    mesh=_MESH,
    out_type=jax.ShapeDtypeStruct((NC, NP, D), jnp.float32),
    scratch_types=[
        pltpu.VMEM((ECHUNK,), jnp.int32),
        pltpu.VMEM((ECHUNK,), jnp.int32),
        pltpu.VMEM((ECHUNK, D), jnp.float32),
        pltpu.VMEM((ECHUNK, D), jnp.float32),
        pltpu.VMEM((ZROWS, D), jnp.float32),
        pltpu.VMEM_SHARED((NP, D), jnp.float32),
        pltpu.SemaphoreType.DMA,
        pltpu.SemaphoreType.DMA,
    ],
)
def _scatter(m_hbm, row_hbm, out_hbm, i0, i1, m0, m1, zb_v, acc_sh, s0, s1):
    cid = lax.axis_index("c")
    sid = lax.axis_index("s")
    wid = sid * NC + cid
    tbase = sid * NPT

    zero = jnp.zeros((16,), jnp.float32)

    def zrow(r, carry):
        def zcol(c, carry2):
            zb_v[r, pl.ds(c * 16, 16)] = zero
            return carry2
        return lax.fori_loop(0, D // 16, zcol, carry)

    lax.fori_loop(0, ZROWS, zrow, 0)

    def zcopy(j, carry):
        pltpu.sync_copy(zb_v, acc_sh.at[pl.ds(tbase + j * ZROWS, ZROWS)])
        return carry

    lax.fori_loop(0, NPT // ZROWS, zcopy, 0)
    plsc.subcore_barrier()

    base = wid * EPW

    def issue(c, iv, mv, sem):
        off = base + c * ECHUNK
        pltpu.async_copy(row_hbm.at[pl.ds(off, ECHUNK)], iv, sem)
        pltpu.async_copy(m_hbm.at[pl.ds(off, ECHUNK)], mv, sem)

    def wait_ld(iv, mv, sem):
        pltpu.make_async_copy(row_hbm.at[pl.ds(0, ECHUNK)], iv, sem).wait()
        pltpu.make_async_copy(m_hbm.at[pl.ds(0, ECHUNK)], mv, sem).wait()

    issue(0, i0, m0, s0)

    def body(j, carry):
        issue(2 * j + 1, i1, m1, s1)
        wait_ld(i0, m0, s0)
        pltpu.sync_copy(m0, acc_sh.at[i0], add=True)
        issue(2 * j + 2, i0, m0, s0)
        wait_ld(i1, m1, s1)
        pltpu.sync_copy(m1, acc_sh.at[i1], add=True)
        return carry

    lax.fori_loop(0, (NCHUNKS - 1) // 2, body, 0)
    wait_ld(i0, m0, s0)
    pltpu.sync_copy(m0, acc_sh.at[i0], add=True)
    plsc.subcore_barrier()

    pltpu.sync_copy(acc_sh.at[pl.ds(tbase, NPT)],
                    out_hbm.at[cid, pl.ds(tbase, NPT)])


def kernel(h, edges, coords, edge_attr, params):
    row, col = edges[0], edges[1]
    cpad = jnp.pad(coords, ((0, 0), (0, CD - 3)))
    xs, ys, zs = coords[:, 0], coords[:, 1], coords[:, 2]
    out = h
    for i, p in enumerate(params):
        w1 = p["edge_W1"]
        w1s, w1d = w1[:D], w1[D:2 * D]
        wr = w1[2 * D:2 * D + 1]
        w1e = w1[2 * D + 1:]
        b1 = p["edge_b1"][None, :]
        b2 = p["edge_b2"][None, :]
        ts, td = _tables(out, cpad, w1s, w1d, wr, b1)
        gs = [_gather(ts, td, xs, ys, zs,
                      row[s * ESEG:(s + 1) * ESEG],
                      col[s * ESEG:(s + 1) * ESEG]) for s in range(NSEG)]
        parts = []
        for s in range(NSEG):
            g, cross = gs[s]
            m = _edge(g, cross.reshape(ESEG // BE, 1, BE),
                      edge_attr[s * ESEG:(s + 1) * ESEG], w1e, wr,
                      p["edge_W2"], b2)
            parts.append(_scatter(m, row[s * ESEG:(s + 1) * ESEG]))
        out = _node(out, parts, p["node_W1"], p["node_b1"][None, :],
                    p["node_W2"], p["node_b2"][None, :], residual=(i > 0))
    return out
